# baseline (speedup 1.0000x reference)
import functools

import jax
import jax.numpy as jnp
from jax import lax
from jax.experimental import pallas as pl
from jax.experimental.pallas import tpu as pltpu, tpu_sc as plsc

NUM_SETS = 32
SET_SIZE = 16
D = 128
NUM_GENES_USED = NUM_SETS * SET_SIZE
LANES = 16
DV = D // LANES

BCHUNK = 16
SC_BATCH = 128
TC_GC = 128


def _lane_shuffle(x, idx):
    return jnp.take_along_axis(x, idx, axis=0)


def _sc_body(gene_hbm, attn_hbm, out_hbm, attn_v, slab_a, slab_b, out_v, sem_a, sem_b):
    nc = 2
    wid = lax.axis_index("s") * nc + lax.axis_index("c")
    n_chunks = out_hbm.shape[0] // BCHUNK
    g0 = wid * SET_SIZE

    pltpu.sync_copy(attn_hbm, attn_v)
    av = attn_v[wid, :]
    lane = lax.iota(jnp.int32, LANES)
    m = av
    for sh in (8, 4, 2, 1):
        m = jnp.maximum(m, _lane_shuffle(m, lane ^ sh))
    e = jnp.exp(av - m)
    tot = e
    for sh in (8, 4, 2, 1):
        tot = tot + _lane_shuffle(tot, lane ^ sh)
    wvec = e / tot

    g_src = gene_hbm.at[pl.ds(g0, SET_SIZE)]

    def start_chunk(c, buf, sem):
        pltpu.make_async_copy(
            g_src.at[:, pl.ds(c * BCHUNK, BCHUNK), :], buf, sem
        ).start()

    def compute_chunk(buf, c):
        def b_body(bl, _):
            accs = [jnp.zeros((LANES,), jnp.float32) for _ in range(DV)]
            for k in range(SET_SIZE):
                wk = wvec[k]
                for v in range(DV):
                    accs[v] = accs[v] + wk * buf[k, bl, pl.ds(v * LANES, LANES)]
            for v in range(DV):
                out_v[bl, pl.ds(v * LANES, LANES)] = accs[v]
            return 0

        lax.fori_loop(0, BCHUNK, b_body, 0)
        pltpu.sync_copy(out_v, out_hbm.at[pl.ds(c * BCHUNK, BCHUNK), wid, :])

    start_chunk(0, slab_a, sem_a)

    def pair_body(i, _):
        c0 = 2 * i
        start_chunk(c0 + 1, slab_b, sem_b)
        pltpu.make_async_copy(
            g_src.at[:, pl.ds(c0 * BCHUNK, BCHUNK), :], slab_a, sem_a
        ).wait()
        compute_chunk(slab_a, c0)

        @pl.when(c0 + 2 < n_chunks)
        def _():
            start_chunk(c0 + 2, slab_a, sem_a)

        pltpu.make_async_copy(
            g_src.at[:, pl.ds((c0 + 1) * BCHUNK, BCHUNK), :], slab_b, sem_b
        ).wait()
        compute_chunk(slab_b, c0 + 1)
        return 0

    lax.fori_loop(0, n_chunks // 2, pair_body, 0)

    if n_chunks % 2:
        c_last = n_chunks - 1
        pltpu.make_async_copy(
            g_src.at[:, pl.ds(c_last * BCHUNK, BCHUNK), :], slab_a, sem_a
        ).wait()
        compute_chunk(slab_a, c_last)


def _tc_pool_body(x_ref, a_ref, o_ref):
    ns = TC_GC // SET_SIZE
    n_tc = x_ref.shape[1]
    a = a_ref[...]
    m = jnp.max(a, axis=1, keepdims=True)
    e = jnp.exp(a - m)
    w = e / jnp.sum(e, axis=1, keepdims=True)
    x4 = x_ref[...].reshape(ns, SET_SIZE, n_tc, D)
    y = jnp.sum(x4 * w[:, :, None, None], axis=1)
    o_ref[...] = jnp.swapaxes(y, 0, 1)


def kernel(gene_features, attn_w):
    b = gene_features.shape[0]
    n_tc = b - SC_BATCH
    attn2 = attn_w.reshape(NUM_SETS, SET_SIZE)
    gf_t = jnp.transpose(gene_features, (1, 0, 2))

    mesh = plsc.VectorSubcoreMesh(core_axis_name="c", subcore_axis_name="s")
    f = pl.kernel(
        _sc_body,
        out_type=jax.ShapeDtypeStruct((SC_BATCH, NUM_SETS, D), jnp.float32),
        mesh=mesh,
        scratch_types=[
            pltpu.VMEM((NUM_SETS, SET_SIZE), jnp.float32),
            pltpu.VMEM((SET_SIZE, BCHUNK, D), jnp.float32),
            pltpu.VMEM((SET_SIZE, BCHUNK, D), jnp.float32),
            pltpu.VMEM((BCHUNK, D), jnp.float32),
            pltpu.SemaphoreType.DMA,
            pltpu.SemaphoreType.DMA,
        ],
    )
    out_sc = f(gf_t, attn2)

    assert SC_BATCH % n_tc == 0 or n_tc % SC_BATCH == 0 or SC_BATCH == n_tc
    ns = TC_GC // SET_SIZE
    out_tc = pl.pallas_call(
        _tc_pool_body,
        grid=(NUM_GENES_USED // TC_GC,),
        in_specs=[
            pl.BlockSpec((TC_GC, n_tc, D), lambda g: (g, SC_BATCH // n_tc, 0)),
            pl.BlockSpec((TC_GC // SET_SIZE, SET_SIZE), lambda g: (g, 0)),
        ],
        out_specs=pl.BlockSpec((n_tc, ns, D), lambda g: (0, g, 0)),
        out_shape=jax.ShapeDtypeStruct((n_tc, NUM_SETS, D), jnp.float32),
    )(gf_t, attn2)

    return jnp.concatenate([out_sc, out_tc], axis=0)


# speedup vs baseline: 1.0465x; 1.0465x over previous
import functools

import jax
import jax.numpy as jnp
from jax import lax
from jax.experimental import pallas as pl
from jax.experimental.pallas import tpu as pltpu, tpu_sc as plsc

NUM_SETS = 32
SET_SIZE = 16
D = 128
NUM_GENES_USED = NUM_SETS * SET_SIZE
LANES = 16
DV = D // LANES

BCHUNK = 16
SC_BATCH = 64
TC_GC = 128


def _lane_shuffle(x, idx):
    return jnp.take_along_axis(x, idx, axis=0)


def _sc_body(gene_hbm, attn_hbm, out_hbm, attn_v, slab_a, slab_b, out_v, sem_a, sem_b):
    nc = 2
    wid = lax.axis_index("s") * nc + lax.axis_index("c")
    n_chunks = out_hbm.shape[0] // BCHUNK
    g0 = wid * SET_SIZE

    pltpu.sync_copy(attn_hbm, attn_v)
    av = attn_v[wid, :]
    lane = lax.iota(jnp.int32, LANES)
    m = av
    for sh in (8, 4, 2, 1):
        m = jnp.maximum(m, _lane_shuffle(m, lane ^ sh))
    e = jnp.exp(av - m)
    tot = e
    for sh in (8, 4, 2, 1):
        tot = tot + _lane_shuffle(tot, lane ^ sh)
    wvec = e / tot

    b0 = gene_hbm.shape[1] - n_chunks * BCHUNK
    g_src = gene_hbm.at[pl.ds(g0, SET_SIZE)]

    def start_chunk(c, buf, sem):
        pltpu.make_async_copy(
            g_src.at[:, pl.ds(b0 + c * BCHUNK, BCHUNK), :], buf, sem
        ).start()

    def compute_chunk(buf, c):
        def b_body(bl, _):
            accs = [jnp.zeros((LANES,), jnp.float32) for _ in range(DV)]
            for k in range(SET_SIZE):
                wk = wvec[k]
                for v in range(DV):
                    accs[v] = accs[v] + wk * buf[k, bl, pl.ds(v * LANES, LANES)]
            for v in range(DV):
                out_v[bl, pl.ds(v * LANES, LANES)] = accs[v]
            return 0

        lax.fori_loop(0, BCHUNK, b_body, 0)
        pltpu.sync_copy(out_v, out_hbm.at[pl.ds(c * BCHUNK, BCHUNK), wid, :])

    start_chunk(0, slab_a, sem_a)

    def pair_body(i, _):
        c0 = 2 * i
        start_chunk(c0 + 1, slab_b, sem_b)
        pltpu.make_async_copy(
            g_src.at[:, pl.ds(b0 + c0 * BCHUNK, BCHUNK), :], slab_a, sem_a
        ).wait()
        compute_chunk(slab_a, c0)

        @pl.when(c0 + 2 < n_chunks)
        def _():
            start_chunk(c0 + 2, slab_a, sem_a)

        pltpu.make_async_copy(
            g_src.at[:, pl.ds(b0 + (c0 + 1) * BCHUNK, BCHUNK), :], slab_b, sem_b
        ).wait()
        compute_chunk(slab_b, c0 + 1)
        return 0

    lax.fori_loop(0, n_chunks // 2, pair_body, 0)

    if n_chunks % 2:
        c_last = n_chunks - 1
        pltpu.make_async_copy(
            g_src.at[:, pl.ds(b0 + c_last * BCHUNK, BCHUNK), :], slab_a, sem_a
        ).wait()
        compute_chunk(slab_a, c_last)


def _tc_pool_body(x_ref, a_ref, o_ref):
    ns = TC_GC // SET_SIZE
    n_tc = x_ref.shape[1]
    a = a_ref[...]
    m = jnp.max(a, axis=1, keepdims=True)
    e = jnp.exp(a - m)
    w = e / jnp.sum(e, axis=1, keepdims=True)
    x4 = x_ref[...].reshape(ns, SET_SIZE, n_tc, D)
    y = jnp.sum(x4 * w[:, :, None, None], axis=1)
    o_ref[...] = jnp.swapaxes(y, 0, 1)


def kernel(gene_features, attn_w):
    b = gene_features.shape[0]
    n_tc = b - SC_BATCH
    attn2 = attn_w.reshape(NUM_SETS, SET_SIZE)
    gf_t = jnp.transpose(gene_features, (1, 0, 2))

    mesh = plsc.VectorSubcoreMesh(core_axis_name="c", subcore_axis_name="s")
    f = pl.kernel(
        _sc_body,
        out_type=jax.ShapeDtypeStruct((SC_BATCH, NUM_SETS, D), jnp.float32),
        mesh=mesh,
        scratch_types=[
            pltpu.VMEM((NUM_SETS, SET_SIZE), jnp.float32),
            pltpu.VMEM((SET_SIZE, BCHUNK, D), jnp.float32),
            pltpu.VMEM((SET_SIZE, BCHUNK, D), jnp.float32),
            pltpu.VMEM((BCHUNK, D), jnp.float32),
            pltpu.SemaphoreType.DMA,
            pltpu.SemaphoreType.DMA,
        ],
    )
    out_sc = f(gf_t, attn2)

    ns = TC_GC // SET_SIZE
    out_tc = pl.pallas_call(
        _tc_pool_body,
        grid=(NUM_GENES_USED // TC_GC,),
        in_specs=[
            pl.BlockSpec((TC_GC, n_tc, D), lambda g: (g, 0, 0)),
            pl.BlockSpec((TC_GC // SET_SIZE, SET_SIZE), lambda g: (g, 0)),
        ],
        out_specs=pl.BlockSpec((n_tc, ns, D), lambda g: (0, g, 0)),
        out_shape=jax.ShapeDtypeStruct((n_tc, NUM_SETS, D), jnp.float32),
    )(gf_t, attn2)

    return jnp.concatenate([out_tc, out_sc], axis=0)
